# initial kernel scaffold (unmeasured)
import jax
import jax.numpy as jnp
from jax import lax
from jax.experimental import pallas as pl
from jax.experimental.pallas import tpu as pltpu

N_DEV = 32
M = 2048
N = 2048
CHUNK = M // N_DEV


def kernel(A, B):
    def body(a_ref, b_ref, out_ref, recv_buf, rs_send, rs_recv, ag_send, ag_recv):
        my = lax.axis_index("i")
        left = jnp.mod(my - 1, N_DEV)
        right = jnp.mod(my + 1, N_DEV)

        barrier = pltpu.get_barrier_semaphore()
        pl.semaphore_signal(
            barrier, inc=1, device_id=(left,), device_id_type=pl.DeviceIdType.MESH
        )
        pl.semaphore_signal(
            barrier, inc=1, device_id=(right,), device_id_type=pl.DeviceIdType.MESH
        )
        pl.semaphore_wait(barrier, 2)

        out_ref[...] = jnp.dot(
            a_ref[...], b_ref[...], preferred_element_type=jnp.float32
        )

        for s in range(N_DEV - 1):
            c_send = jnp.mod(my - s, N_DEV)
            rdma = pltpu.make_async_remote_copy(
                src_ref=out_ref.at[pl.ds(c_send * CHUNK, CHUNK), :],
                dst_ref=recv_buf.at[s],
                send_sem=rs_send.at[s],
                recv_sem=rs_recv.at[s],
                device_id=(right,),
                device_id_type=pl.DeviceIdType.MESH,
            )
            rdma.start()
            rdma.wait()
            c_recv = jnp.mod(my - s - 1, N_DEV)
            idx = pl.ds(c_recv * CHUNK, CHUNK)
            out_ref[idx, :] += recv_buf[s]

        g = jnp.mod(my + 1, N_DEV)
        gidx = pl.ds(g * CHUNK, CHUNK)
        z = out_ref[gidx, :]
        out_ref[gidx, :] = (
            0.5 * z * (1.0 + jnp.tanh(0.7978845608 * (z + 0.044715 * z * z * z)))
        )

        for t in range(N_DEV - 1):
            c_fwd = jnp.mod(my + 1 - t, N_DEV)
            idx = pl.ds(c_fwd * CHUNK, CHUNK)
            rdma = pltpu.make_async_remote_copy(
                src_ref=out_ref.at[idx, :],
                dst_ref=out_ref.at[idx, :],
                send_sem=ag_send.at[t],
                recv_sem=ag_recv.at[t],
                device_id=(right,),
                device_id_type=pl.DeviceIdType.MESH,
            )
            rdma.start()
            rdma.wait()

    return pl.pallas_call(
        body,
        out_shape=jax.ShapeDtypeStruct((M, N), jnp.float32),
        in_specs=[
            pl.BlockSpec(memory_space=pltpu.VMEM),
            pl.BlockSpec(memory_space=pltpu.VMEM),
        ],
        out_specs=pl.BlockSpec(memory_space=pltpu.VMEM),
        scratch_shapes=[
            pltpu.VMEM((N_DEV - 1, CHUNK, N), jnp.float32),
            pltpu.SemaphoreType.DMA((N_DEV - 1,)),
            pltpu.SemaphoreType.DMA((N_DEV - 1,)),
            pltpu.SemaphoreType.DMA((N_DEV - 1,)),
            pltpu.SemaphoreType.DMA((N_DEV - 1,)),
        ],
        compiler_params=pltpu.CompilerParams(collective_id=0),
    )(A, B)


# baseline (device time: 503357 ns/iter reference)
import jax
import jax.numpy as jnp
from jax import lax
from jax.experimental import pallas as pl
from jax.experimental.pallas import tpu as pltpu

N_DEV = 32
M = 2048
N = 2048
CHUNK = M // N_DEV


def kernel(A, B):
    def body(a_ref, b_ref, out_ref, recv_buf, rs_send, rs_recv, ag_send, ag_recv):
        my = lax.axis_index("i")
        left = jnp.mod(my - 1, N_DEV)
        right = jnp.mod(my + 1, N_DEV)

        barrier = pltpu.get_barrier_semaphore()
        pl.semaphore_signal(
            barrier, inc=1, device_id=(left,), device_id_type=pl.DeviceIdType.MESH
        )
        pl.semaphore_signal(
            barrier, inc=1, device_id=(right,), device_id_type=pl.DeviceIdType.MESH
        )
        pl.semaphore_wait(barrier, 2)

        out_ref[...] = jnp.dot(
            a_ref[...], b_ref[...], preferred_element_type=jnp.float32
        )

        for s in range(N_DEV - 1):
            c_send = jnp.mod(my - s, N_DEV)
            rdma = pltpu.make_async_remote_copy(
                src_ref=out_ref.at[pl.ds(c_send * CHUNK, CHUNK), :],
                dst_ref=recv_buf.at[s],
                send_sem=rs_send.at[s],
                recv_sem=rs_recv.at[s],
                device_id=(right,),
                device_id_type=pl.DeviceIdType.MESH,
            )
            rdma.start()
            rdma.wait()
            c_recv = jnp.mod(my - s - 1, N_DEV)
            idx = pl.ds(c_recv * CHUNK, CHUNK)
            out_ref[idx, :] += recv_buf[s]

        g = jnp.mod(my + 1, N_DEV)
        gidx = pl.ds(g * CHUNK, CHUNK)
        z = out_ref[gidx, :]
        out_ref[gidx, :] = (
            0.5 * z * (1.0 + jnp.tanh(0.7978845608 * (z + 0.044715 * z * z * z)))
        )

        for t in range(N_DEV - 1):
            c_fwd = jnp.mod(my + 1 - t, N_DEV)
            idx = pl.ds(c_fwd * CHUNK, CHUNK)
            rdma = pltpu.make_async_remote_copy(
                src_ref=out_ref.at[idx, :],
                dst_ref=out_ref.at[idx, :],
                send_sem=ag_send.at[t],
                recv_sem=ag_recv.at[t],
                device_id=(right,),
                device_id_type=pl.DeviceIdType.MESH,
            )
            rdma.start()
            rdma.wait()

    return pl.pallas_call(
        body,
        out_shape=jax.ShapeDtypeStruct((M, N), jnp.float32),
        in_specs=[
            pl.BlockSpec(memory_space=pltpu.VMEM),
            pl.BlockSpec(memory_space=pltpu.VMEM),
        ],
        out_specs=pl.BlockSpec(memory_space=pltpu.VMEM),
        scratch_shapes=[
            pltpu.VMEM((N_DEV - 1, CHUNK, N), jnp.float32),
            pltpu.SemaphoreType.DMA((N_DEV - 1,)),
            pltpu.SemaphoreType.DMA((N_DEV - 1,)),
            pltpu.SemaphoreType.DMA((N_DEV - 1,)),
            pltpu.SemaphoreType.DMA((N_DEV - 1,)),
        ],
        compiler_params=pltpu.CompilerParams(
            collective_id=0, vmem_limit_bytes=100 * 1024 * 1024
        ),
    )(A, B)


# device time: 313059 ns/iter; 1.6079x vs baseline; 1.6079x over previous
import jax
import jax.numpy as jnp
from jax import lax
from jax.experimental import pallas as pl
from jax.experimental.pallas import tpu as pltpu

N_DEV = 32
M = 2048
N = 2048
CHUNK = M // N_DEV
HC = N // 2


def _build_ring_tables():
    log_coords = []
    for z in range(4):
        for y in range(4):
            for x in ([0, 1] if y % 2 == 0 else [1, 0]):
                log_coords.append((x, y, z))
    log_of = {c: i for i, c in enumerate(log_coords)}
    P = [(0, 0), (1, 0), (2, 0), (3, 0), (3, 1), (2, 1), (1, 1), (0, 1),
         (0, 2), (1, 2), (2, 2), (3, 2), (3, 3), (2, 3), (1, 3), (0, 3)]
    ring = [(0, y, z) for (y, z) in P] + [(1,) + P[31 - r] for r in range(16, 32)]
    for r in range(N_DEV):
        a, b = ring[r], ring[(r + 1) % N_DEV]
        assert sum(abs(a[k] - b[k]) for k in range(3)) == 1, (r, a, b)
    ring_log = [log_of[c] for c in ring]
    pos_of_log = [0] * N_DEV
    right_of_log = [0] * N_DEV
    left_of_log = [0] * N_DEV
    for r, l in enumerate(ring_log):
        pos_of_log[l] = r
        right_of_log[l] = ring_log[(r + 1) % N_DEV]
        left_of_log[l] = ring_log[(r - 1) % N_DEV]
    return pos_of_log, right_of_log, left_of_log


_POS_OF_LOG, _RIGHT_OF_LOG, _LEFT_OF_LOG = _build_ring_tables()


def _lut(table, idx):
    out = jnp.int32(table[0])
    for k in range(1, N_DEV):
        out = jnp.where(idx == jnp.int32(k), jnp.int32(table[k]), out)
    return out


def _gelu(z):
    return 0.5 * z * (1.0 + jnp.tanh(0.7978845608 * (z + 0.044715 * z * z * z)))


def kernel(A, B):
    def body(
        a_ref, b_ref, out_ref,
        rs_r_buf, rs_l_buf,
        rsr_send, rsr_recv, rsl_send, rsl_recv,
        agr_send, agr_recv, agl_send, agl_recv,
    ):
        my_log = lax.axis_index("i")
        r = _lut(_POS_OF_LOG, my_log)
        right = _lut(_RIGHT_OF_LOG, my_log)
        left = _lut(_LEFT_OF_LOG, my_log)

        def rows(c):
            return pl.ds(c * CHUNK, CHUNK)

        colR = pl.ds(0, HC)
        colL = pl.ds(HC, HC)

        barrier = pltpu.get_barrier_semaphore()
        pl.semaphore_signal(
            barrier, inc=1, device_id=(left,), device_id_type=pl.DeviceIdType.MESH
        )
        pl.semaphore_signal(
            barrier, inc=1, device_id=(right,), device_id_type=pl.DeviceIdType.MESH
        )
        pl.semaphore_wait(barrier, 2)

        c0 = jnp.mod(r, N_DEV)
        out_ref[rows(c0), :] = jnp.dot(
            a_ref[rows(c0), :], b_ref[...], preferred_element_type=jnp.float32
        )

        for s in range(N_DEV - 1):
            c_sr = jnp.mod(r - s, N_DEV)
            c_sl = jnp.mod(r + s, N_DEV)
            rdma_r = pltpu.make_async_remote_copy(
                src_ref=out_ref.at[rows(c_sr), colR],
                dst_ref=rs_r_buf.at[s],
                send_sem=rsr_send.at[s],
                recv_sem=rsr_recv.at[s],
                device_id=(right,),
                device_id_type=pl.DeviceIdType.MESH,
            )
            rdma_l = pltpu.make_async_remote_copy(
                src_ref=out_ref.at[rows(c_sl), colL],
                dst_ref=rs_l_buf.at[s],
                send_sem=rsl_send.at[s],
                recv_sem=rsl_recv.at[s],
                device_id=(left,),
                device_id_type=pl.DeviceIdType.MESH,
            )
            rdma_r.start()
            rdma_l.start()

            c_rr = jnp.mod(r - s - 1, N_DEV)
            c_rl = jnp.mod(r + s + 1, N_DEV)
            out_ref[rows(c_rr), colR] = jnp.dot(
                a_ref[rows(c_rr), :], b_ref[:, colR],
                preferred_element_type=jnp.float32,
            )
            out_ref[rows(c_rl), colL] = jnp.dot(
                a_ref[rows(c_rl), :], b_ref[:, colL],
                preferred_element_type=jnp.float32,
            )

            rdma_r.wait()
            out_ref[rows(c_rr), colR] += rs_r_buf[s]
            rdma_l.wait()
            out_ref[rows(c_rl), colL] += rs_l_buf[s]

        g_r = jnp.mod(r + 1, N_DEV)
        g_l = jnp.mod(r - 1, N_DEV)
        out_ref[rows(g_r), colR] = _gelu(out_ref[rows(g_r), colR])
        out_ref[rows(g_l), colL] = _gelu(out_ref[rows(g_l), colL])

        for t in range(N_DEV - 1):
            c_fr = jnp.mod(r + 1 - t, N_DEV)
            c_fl = jnp.mod(r - 1 + t, N_DEV)
            rdma_r = pltpu.make_async_remote_copy(
                src_ref=out_ref.at[rows(c_fr), colR],
                dst_ref=out_ref.at[rows(c_fr), colR],
                send_sem=agr_send.at[t],
                recv_sem=agr_recv.at[t],
                device_id=(right,),
                device_id_type=pl.DeviceIdType.MESH,
            )
            rdma_l = pltpu.make_async_remote_copy(
                src_ref=out_ref.at[rows(c_fl), colL],
                dst_ref=out_ref.at[rows(c_fl), colL],
                send_sem=agl_send.at[t],
                recv_sem=agl_recv.at[t],
                device_id=(left,),
                device_id_type=pl.DeviceIdType.MESH,
            )
            rdma_r.start()
            rdma_l.start()
            rdma_r.wait()
            rdma_l.wait()

    nsem = N_DEV - 1
    return pl.pallas_call(
        body,
        out_shape=jax.ShapeDtypeStruct((M, N), jnp.float32),
        in_specs=[
            pl.BlockSpec(memory_space=pltpu.VMEM),
            pl.BlockSpec(memory_space=pltpu.VMEM),
        ],
        out_specs=pl.BlockSpec(memory_space=pltpu.VMEM),
        scratch_shapes=[
            pltpu.VMEM((nsem, CHUNK, HC), jnp.float32),
            pltpu.VMEM((nsem, CHUNK, HC), jnp.float32),
            pltpu.SemaphoreType.DMA((nsem,)),
            pltpu.SemaphoreType.DMA((nsem,)),
            pltpu.SemaphoreType.DMA((nsem,)),
            pltpu.SemaphoreType.DMA((nsem,)),
            pltpu.SemaphoreType.DMA((nsem,)),
            pltpu.SemaphoreType.DMA((nsem,)),
            pltpu.SemaphoreType.DMA((nsem,)),
            pltpu.SemaphoreType.DMA((nsem,)),
        ],
        compiler_params=pltpu.CompilerParams(
            collective_id=0, vmem_limit_bytes=100 * 1024 * 1024
        ),
    )(A, B)
